# baseline (speedup 1.0000x reference)
import functools
import math

import jax
import jax.numpy as jnp
from jax import lax
from jax.experimental import pallas as pl
from jax.experimental.pallas import tpu as pltpu
from jax.experimental.pallas import tpu_sc as plsc

N = 10000
E = 320000
HID = 128
G = 64
OUT_CH = 10

NC = 2
NS = 16
NW = NC * NS
EPW = E // NW
C = 80
NCHUNK = EPW // C
L = 16
RG = C // L
INV_SQRT_D = 1.0 / math.sqrt(HID)

_mesh = plsc.VectorSubcoreMesh(
    core_axis_name="c", subcore_axis_name="s", num_cores=NC, num_subcores=NS)
_SC_PARAMS = pltpu.CompilerParams(needs_layout_passes=False)


def _proj(x, Wq, bq, Wk, bk, Wv, bv, Ws, bs):
    q = jnp.dot(x, Wq, preferred_element_type=jnp.float32) + bq
    k = jnp.dot(x, Wk, preferred_element_type=jnp.float32) + bk
    v = jnp.dot(x, Wv, preferred_element_type=jnp.float32) + bv
    s = jnp.dot(x, Ws, preferred_element_type=jnp.float32) + bs
    return q, k, v, s


def _write_proj(q_ref, kv_ref, skip_ref, q, k, v, s):
    q_ref[...] = q
    kv_ref[:, 0:HID] = k
    kv_ref[:, HID:2 * HID] = v
    skip_ref[...] = s


def _dense1_body(x_ref, Wq_ref, bq_ref, Wk_ref, bk_ref, Wv_ref, bv_ref,
                 Ws_ref, bs_ref,
                 q_ref, kv_ref, skip_ref):
    out = _proj(x_ref[...], Wq_ref[...], bq_ref[...], Wk_ref[...], bk_ref[...],
                Wv_ref[...], bv_ref[...], Ws_ref[...], bs_ref[...])
    _write_proj(q_ref, kv_ref, skip_ref, *out)


def _combine(aggp, sp, skip):
    s = (sp[0] + sp[1]).reshape(N, 1)
    h = (aggp[0] + aggp[1]) / (s + 1e-16) + skip
    return jnp.maximum(h, 0.0)


def _dense2_body(aggp_ref, sp_ref, skip1_ref,
                 Wq_ref, bq_ref, Wk_ref, bk_ref, Wv_ref, bv_ref,
                 Ws_ref, bs_ref,
                 q_ref, kv_ref, skip_ref):
    h = _combine(aggp_ref[...], sp_ref[...], skip1_ref[...])
    out = _proj(h, Wq_ref[...], bq_ref[...], Wk_ref[...], bk_ref[...],
                Wv_ref[...], bv_ref[...], Ws_ref[...], bs_ref[...])
    _write_proj(q_ref, kv_ref, skip_ref, *out)


def _final_body(aggp_ref, sp_ref, skip2_ref, batch_ref,
                Wl_ref, bl_ref, out_ref):
    h = _combine(aggp_ref[...], sp_ref[...], skip2_ref[...])
    seg = lax.broadcasted_iota(jnp.int32, (G, N), 0)
    oh = (seg == batch_ref[...]).astype(jnp.float32)
    cnt = jnp.sum(oh, axis=1, keepdims=True)
    sums = jnp.dot(oh, h, preferred_element_type=jnp.float32)
    mean = sums / jnp.maximum(cnt, 1.0)
    out_ref[...] = jnp.dot(mean, Wl_ref[...],
                           preferred_element_type=jnp.float32) + bl_ref[...]


_PROJ_OUT = [jax.ShapeDtypeStruct((N, HID), jnp.float32),
             jax.ShapeDtypeStruct((N, 2 * HID), jnp.float32),
             jax.ShapeDtypeStruct((N, HID), jnp.float32)]

_dense1 = pl.pallas_call(_dense1_body, out_shape=_PROJ_OUT)
_dense2 = pl.pallas_call(_dense2_body, out_shape=_PROJ_OUT)
_final = pl.pallas_call(_final_body,
                        out_shape=jax.ShapeDtypeStruct((G, OUT_CH), jnp.float32))


def _edge_body(q_hbm, kv_hbm, we_hbm, edata_hbm,
               agg_out, s_out,
               idxs_v, edata_v, src_i, dst_i, ew_i, qrows, kvrows, sbuf, part,
               p_v, we_v, agg_sh, s_sh, sem):
    cid = lax.axis_index("c")
    sid = lax.axis_index("s")
    wid = cid * NS + sid

    def zrow_body(r, carry):
        for j in range(HID // L):
            sbuf[r, pl.ds(j * L, L)] = jnp.zeros((L,), jnp.float32)
        return carry

    lax.fori_loop(0, C, zrow_body, 0)
    for g in range(RG):
        p_v[pl.ds(g * L, L)] = jnp.zeros((L,), jnp.float32)

    def zchunk_body(kk, carry):
        off = pl.multiple_of((kk * NS + sid) * C, 8)
        pltpu.sync_copy(sbuf, agg_sh.at[pl.ds(off, C)])
        pltpu.sync_copy(p_v, s_sh.at[pl.ds(off, C)])
        return carry

    lax.fori_loop(0, N // C // NS, zchunk_body, 0)
    @pl.when(sid < (N // C) % NS)
    def _():
        off = pl.multiple_of(((N // C // NS) * NS + sid) * C, 8)
        pltpu.sync_copy(sbuf, agg_sh.at[pl.ds(off, C)])
        pltpu.sync_copy(p_v, s_sh.at[pl.ds(off, C)])
    plsc.subcore_barrier()

    pltpu.sync_copy(we_hbm, we_v)

    iota = lax.iota(jnp.int32, L)
    iota16 = iota * L
    row0 = wid * NCHUNK
    wevs = [we_v[pl.ds(j * L, L)] for j in range(HID // L)]

    def super_body(ss, carry):
        idxs_v[...] = jnp.full((L,), row0 + ss * L, jnp.int32) + iota
        pltpu.async_copy(edata_hbm.at[idxs_v], edata_v, sem).wait()
        nin = jnp.minimum(L, NCHUNK - ss * L)

        def chunk_body(k, carry1):
            for g in range(RG):
                src_i[pl.ds(g * L, L)] = edata_v[k, pl.ds(g * L, L)]
                dst_i[pl.ds(g * L, L)] = edata_v[k, pl.ds(C + g * L, L)]
                ew_i[pl.ds(g * L, L)] = edata_v[k, pl.ds(2 * C + g * L, L)]
            c1 = pltpu.async_copy(q_hbm.at[dst_i], qrows, sem)
            c2 = pltpu.async_copy(kv_hbm.at[src_i], kvrows, sem)
            c1.wait()
            c2.wait()

            def group_body(g, carry2):
                r0 = g * L
                for t in range(L):
                    r = r0 + t
                    aew = plsc.bitcast(
                        plsc.load_gather(ew_i, [jnp.full((L,), r, jnp.int32)]),
                        jnp.float32)
                    acc = qrows[r, pl.ds(0, L)] * (kvrows[r, pl.ds(0, L)]
                                                   + aew * wevs[0])
                    for j in range(1, HID // L):
                        acc = acc + qrows[r, pl.ds(j * L, L)] * (
                            kvrows[r, pl.ds(j * L, L)] + aew * wevs[j])
                    part[pl.ds(t * L, L)] = acc
                tot = plsc.load_gather(part, [iota16])
                for j in range(1, L):
                    tot = tot + plsc.load_gather(part, [iota16 + j])
                p16 = jnp.exp(tot * INV_SQRT_D)
                p_v[pl.ds(r0, L)] = p16
                for t in range(L):
                    r = r0 + t
                    rsplat = jnp.full((L,), r, jnp.int32)
                    ap = plsc.load_gather(p_v, [rsplat])
                    apw = ap * plsc.bitcast(plsc.load_gather(ew_i, [rsplat]),
                                            jnp.float32)
                    for j in range(HID // L):
                        sbuf[r, pl.ds(j * L, L)] = (
                            kvrows[r, pl.ds(HID + j * L, L)] * ap
                            + apw * wevs[j])
                return carry2

            lax.fori_loop(0, RG, group_body, 0)
            s1 = pltpu.async_copy(sbuf, agg_sh.at[dst_i], sem, add=True)
            s2 = pltpu.async_copy(p_v, s_sh.at[dst_i], sem, add=True)
            s1.wait()
            s2.wait()
            return carry1

        lax.fori_loop(0, nin, chunk_body, 0)
        return carry

    lax.fori_loop(0, (NCHUNK + L - 1) // L, super_body, 0)
    plsc.subcore_barrier()

    @pl.when(sid == 0)
    def _():
        pltpu.sync_copy(agg_sh, agg_out.at[cid])
        pltpu.sync_copy(s_sh, s_out.at[cid, 0])


_edge_pass = pl.kernel(
    _edge_body,
    out_type=(jax.ShapeDtypeStruct((NC, N, HID), jnp.float32),
              jax.ShapeDtypeStruct((NC, 1, N), jnp.float32)),
    mesh=_mesh,
    compiler_params=_SC_PARAMS,
    scratch_types=(
        pltpu.VMEM((L,), jnp.int32),
        pltpu.VMEM((L, 256), jnp.int32),
        pltpu.VMEM((C,), jnp.int32),
        pltpu.VMEM((C,), jnp.int32),
        pltpu.VMEM((C,), jnp.int32),
        pltpu.VMEM((C, HID), jnp.float32),
        pltpu.VMEM((C, 2 * HID), jnp.float32),
        pltpu.VMEM((C, HID), jnp.float32),
        pltpu.VMEM((L * L,), jnp.float32),
        pltpu.VMEM((C,), jnp.float32),
        pltpu.VMEM((HID,), jnp.float32),
        pltpu.VMEM_SHARED((N, HID), jnp.float32),
        pltpu.VMEM_SHARED((N,), jnp.float32),
        pltpu.SemaphoreType.DMA,
    ),
)


def kernel(x, edge_index, edge_weight, batch,
           Wq1, bq1, Wk1, bk1, Wv1, bv1, We1, Ws1, bs1,
           Wq2, bq2, Wk2, bk2, Wv2, bv2, We2, Ws2, bs2,
           Wl, bl):
    edata = jnp.concatenate(
        [edge_index[0].reshape(E // C, C),
         edge_index[1].reshape(E // C, C),
         lax.bitcast_convert_type(edge_weight, jnp.int32).reshape(E // C, C),
         jnp.zeros((E // C, 256 - 3 * C), jnp.int32)], axis=1)
    edata = jnp.pad(edata, ((0, 16), (0, 0)))

    q1, kv1, skip1 = _dense1(
        x, Wq1, bq1.reshape(1, HID), Wk1, bk1.reshape(1, HID),
        Wv1, bv1.reshape(1, HID), Ws1, bs1.reshape(1, HID))
    agg1, s1 = _edge_pass(q1, kv1, We1.reshape(HID), edata)

    q2, kv2, skip2 = _dense2(
        agg1, s1[:, 0, :N], skip1,
        Wq2, bq2.reshape(1, HID), Wk2, bk2.reshape(1, HID),
        Wv2, bv2.reshape(1, HID), Ws2, bs2.reshape(1, HID))
    agg2, s2 = _edge_pass(q2, kv2, We2.reshape(HID), edata)

    return _final(agg2, s2[:, 0, :N], skip2,
                  batch.reshape(1, N), Wl, bl.reshape(1, OUT_CH))


# speedup vs baseline: 1.1026x; 1.1026x over previous
import functools
import math

import jax
import jax.numpy as jnp
from jax import lax
from jax.experimental import pallas as pl
from jax.experimental.pallas import tpu as pltpu
from jax.experimental.pallas import tpu_sc as plsc

N = 10000
E = 320000
HID = 128
G = 64
OUT_CH = 10

NC = 2
NS = 16
NW = NC * NS
EPW = E // NW
C = 80
NCHUNK = EPW // C
L = 16
RG = C // L
INV_SQRT_D = 1.0 / math.sqrt(HID)

_mesh = plsc.VectorSubcoreMesh(
    core_axis_name="c", subcore_axis_name="s", num_cores=NC, num_subcores=NS)
_SC_PARAMS = pltpu.CompilerParams(needs_layout_passes=False)


def _proj(x, Wq, bq, Wk, bk, Wv, bv, Ws, bs):
    q = jnp.dot(x, Wq, preferred_element_type=jnp.float32) + bq
    k = jnp.dot(x, Wk, preferred_element_type=jnp.float32) + bk
    v = jnp.dot(x, Wv, preferred_element_type=jnp.float32) + bv
    s = jnp.dot(x, Ws, preferred_element_type=jnp.float32) + bs
    return q, k, v, s


def _write_proj(q_ref, kv_ref, skip_ref, q, k, v, s):
    q_ref[...] = q
    kv_ref[:, 0:HID] = k
    kv_ref[:, HID:2 * HID] = v
    skip_ref[...] = s


def _dense1_body(x_ref, Wq_ref, bq_ref, Wk_ref, bk_ref, Wv_ref, bv_ref,
                 Ws_ref, bs_ref,
                 q_ref, kv_ref, skip_ref):
    out = _proj(x_ref[...], Wq_ref[...], bq_ref[...], Wk_ref[...], bk_ref[...],
                Wv_ref[...], bv_ref[...], Ws_ref[...], bs_ref[...])
    _write_proj(q_ref, kv_ref, skip_ref, *out)


def _combine(aggp, sp, awp, Werow, skip):
    s = (sp[0] + sp[1]).reshape(N, 1)
    aw = (awp[0] + awp[1]).reshape(N, 1)
    h = (aggp[0] + aggp[1] + aw * Werow) / (s + 1e-16) + skip
    return jnp.maximum(h, 0.0)


def _dense2_body(aggp_ref, sp_ref, awp_ref, skip1_ref, Werow_ref,
                 Wq_ref, bq_ref, Wk_ref, bk_ref, Wv_ref, bv_ref,
                 Ws_ref, bs_ref,
                 q_ref, kv_ref, skip_ref):
    h = _combine(aggp_ref[...], sp_ref[...], awp_ref[...], Werow_ref[...],
                 skip1_ref[...])
    out = _proj(h, Wq_ref[...], bq_ref[...], Wk_ref[...], bk_ref[...],
                Wv_ref[...], bv_ref[...], Ws_ref[...], bs_ref[...])
    _write_proj(q_ref, kv_ref, skip_ref, *out)


def _final_body(aggp_ref, sp_ref, awp_ref, skip2_ref, Werow_ref, batch_ref,
                Wl_ref, bl_ref, out_ref):
    h = _combine(aggp_ref[...], sp_ref[...], awp_ref[...], Werow_ref[...],
                 skip2_ref[...])
    seg = lax.broadcasted_iota(jnp.int32, (G, N), 0)
    oh = (seg == batch_ref[...]).astype(jnp.float32)
    cnt = jnp.sum(oh, axis=1, keepdims=True)
    sums = jnp.dot(oh, h, preferred_element_type=jnp.float32)
    mean = sums / jnp.maximum(cnt, 1.0)
    out_ref[...] = jnp.dot(mean, Wl_ref[...],
                           preferred_element_type=jnp.float32) + bl_ref[...]


_PROJ_OUT = [jax.ShapeDtypeStruct((N, HID), jnp.float32),
             jax.ShapeDtypeStruct((N, 2 * HID), jnp.float32),
             jax.ShapeDtypeStruct((N, HID), jnp.float32)]

_dense1 = pl.pallas_call(_dense1_body, out_shape=_PROJ_OUT)
_dense2 = pl.pallas_call(_dense2_body, out_shape=_PROJ_OUT)
_final = pl.pallas_call(_final_body,
                        out_shape=jax.ShapeDtypeStruct((G, OUT_CH), jnp.float32))


def _edge_body(q_hbm, kv_hbm, we_hbm, edata_hbm,
               agg_out, s_out, aw_out,
               idxs_v, edata_v, src_i, dst_i, ew_i, qrows, kvrows, sbuf, part,
               p_v, pw_v, we_v, agg_sh, s_sh, aw_sh, sem):
    cid = lax.axis_index("c")
    sid = lax.axis_index("s")
    wid = cid * NS + sid

    def zrow_body(r, carry):
        for j in range(HID // L):
            sbuf[r, pl.ds(j * L, L)] = jnp.zeros((L,), jnp.float32)
        return carry

    lax.fori_loop(0, C, zrow_body, 0)
    for g in range(RG):
        p_v[pl.ds(g * L, L)] = jnp.zeros((L,), jnp.float32)

    def zchunk_body(kk, carry):
        off = pl.multiple_of((kk * NS + sid) * C, 8)
        pltpu.sync_copy(sbuf, agg_sh.at[pl.ds(off, C)])
        pltpu.sync_copy(p_v, s_sh.at[pl.ds(off, C)])
        pltpu.sync_copy(p_v, aw_sh.at[pl.ds(off, C)])
        return carry

    lax.fori_loop(0, N // C // NS, zchunk_body, 0)
    @pl.when(sid < (N // C) % NS)
    def _():
        off = pl.multiple_of(((N // C // NS) * NS + sid) * C, 8)
        pltpu.sync_copy(sbuf, agg_sh.at[pl.ds(off, C)])
        pltpu.sync_copy(p_v, s_sh.at[pl.ds(off, C)])
        pltpu.sync_copy(p_v, aw_sh.at[pl.ds(off, C)])
    plsc.subcore_barrier()

    pltpu.sync_copy(we_hbm, we_v)

    iota = lax.iota(jnp.int32, L)
    iota16 = iota * L
    row0 = wid * NCHUNK
    wevs = [we_v[pl.ds(j * L, L)] for j in range(HID // L)]

    def super_body(ss, carry):
        idxs_v[...] = jnp.full((L,), row0 + ss * L, jnp.int32) + iota
        pltpu.async_copy(edata_hbm.at[idxs_v], edata_v, sem).wait()
        nin = jnp.minimum(L, NCHUNK - ss * L)

        def chunk_body(k, carry1):
            for g in range(RG):
                src_i[pl.ds(g * L, L)] = edata_v[k, pl.ds(g * L, L)]
                dst_i[pl.ds(g * L, L)] = edata_v[k, pl.ds(C + g * L, L)]
                ew_i[pl.ds(g * L, L)] = edata_v[k, pl.ds(2 * C + g * L, L)]
            c1 = pltpu.async_copy(q_hbm.at[dst_i], qrows, sem)
            c2 = pltpu.async_copy(kv_hbm.at[src_i], kvrows, sem)
            c1.wait()
            c2.wait()

            def group_body(g, carry2):
                r0 = g * L
                for t in range(L):
                    r = r0 + t
                    aew = plsc.bitcast(
                        plsc.load_gather(ew_i, [jnp.full((L,), r, jnp.int32)]),
                        jnp.float32)
                    acc = qrows[r, pl.ds(0, L)] * (kvrows[r, pl.ds(0, L)]
                                                   + aew * wevs[0])
                    for j in range(1, HID // L):
                        acc = acc + qrows[r, pl.ds(j * L, L)] * (
                            kvrows[r, pl.ds(j * L, L)] + aew * wevs[j])
                    part[pl.ds(t * L, L)] = acc
                tot = plsc.load_gather(part, [iota16])
                for j in range(1, L):
                    tot = tot + plsc.load_gather(part, [iota16 + j])
                ewg = plsc.bitcast(ew_i[pl.ds(r0, L)], jnp.float32)
                p16 = jnp.exp(tot * INV_SQRT_D)
                p_v[pl.ds(r0, L)] = p16
                pw_v[pl.ds(r0, L)] = p16 * ewg
                for t in range(L):
                    r = r0 + t
                    ap = plsc.load_gather(p_v, [jnp.full((L,), r, jnp.int32)])
                    for j in range(HID // L):
                        sbuf[r, pl.ds(j * L, L)] = kvrows[r, pl.ds(HID + j * L, L)] * ap
                return carry2

            lax.fori_loop(0, RG, group_body, 0)
            s1 = pltpu.async_copy(sbuf, agg_sh.at[dst_i], sem, add=True)
            s2 = pltpu.async_copy(p_v, s_sh.at[dst_i], sem, add=True)
            s3 = pltpu.async_copy(pw_v, aw_sh.at[dst_i], sem, add=True)
            s1.wait()
            s2.wait()
            s3.wait()
            return carry1

        lax.fori_loop(0, nin, chunk_body, 0)
        return carry

    lax.fori_loop(0, (NCHUNK + L - 1) // L, super_body, 0)
    plsc.subcore_barrier()

    @pl.when(sid == 0)
    def _():
        pltpu.sync_copy(agg_sh, agg_out.at[cid])
        pltpu.sync_copy(s_sh, s_out.at[cid, 0])
        pltpu.sync_copy(aw_sh, aw_out.at[cid, 0])


_edge_pass = pl.kernel(
    _edge_body,
    out_type=(jax.ShapeDtypeStruct((NC, N, HID), jnp.float32),
              jax.ShapeDtypeStruct((NC, 1, N), jnp.float32),
              jax.ShapeDtypeStruct((NC, 1, N), jnp.float32)),
    mesh=_mesh,
    compiler_params=_SC_PARAMS,
    scratch_types=(
        pltpu.VMEM((L,), jnp.int32),
        pltpu.VMEM((L, 256), jnp.int32),
        pltpu.VMEM((C,), jnp.int32),
        pltpu.VMEM((C,), jnp.int32),
        pltpu.VMEM((C,), jnp.int32),
        pltpu.VMEM((C, HID), jnp.float32),
        pltpu.VMEM((C, 2 * HID), jnp.float32),
        pltpu.VMEM((C, HID), jnp.float32),
        pltpu.VMEM((L * L,), jnp.float32),
        pltpu.VMEM((C,), jnp.float32),
        pltpu.VMEM((C,), jnp.float32),
        pltpu.VMEM((HID,), jnp.float32),
        pltpu.VMEM_SHARED((N, HID), jnp.float32),
        pltpu.VMEM_SHARED((N,), jnp.float32),
        pltpu.VMEM_SHARED((N,), jnp.float32),
        pltpu.SemaphoreType.DMA,
    ),
)


def kernel(x, edge_index, edge_weight, batch,
           Wq1, bq1, Wk1, bk1, Wv1, bv1, We1, Ws1, bs1,
           Wq2, bq2, Wk2, bk2, Wv2, bv2, We2, Ws2, bs2,
           Wl, bl):
    edata = jnp.concatenate(
        [edge_index[0].reshape(E // C, C),
         edge_index[1].reshape(E // C, C),
         lax.bitcast_convert_type(edge_weight, jnp.int32).reshape(E // C, C),
         jnp.zeros((E // C, 256 - 3 * C), jnp.int32)], axis=1)
    edata = jnp.pad(edata, ((0, 16), (0, 0)))

    q1, kv1, skip1 = _dense1(
        x, Wq1, bq1.reshape(1, HID), Wk1, bk1.reshape(1, HID),
        Wv1, bv1.reshape(1, HID), Ws1, bs1.reshape(1, HID))
    agg1, s1, aw1 = _edge_pass(q1, kv1, We1.reshape(HID), edata)

    q2, kv2, skip2 = _dense2(
        agg1, s1[:, 0, :N], aw1[:, 0, :N], skip1, We1.reshape(1, HID),
        Wq2, bq2.reshape(1, HID), Wk2, bk2.reshape(1, HID),
        Wv2, bv2.reshape(1, HID), Ws2, bs2.reshape(1, HID))
    agg2, s2, aw2 = _edge_pass(q2, kv2, We2.reshape(HID), edata)

    return _final(agg2, s2[:, 0, :N], aw2[:, 0, :N], skip2,
                  We2.reshape(1, HID),
                  batch.reshape(1, N), Wl, bl.reshape(1, OUT_CH))
